# Initial kernel scaffold; baseline (speedup 1.0000x reference)
import jax
import jax.numpy as jnp
from jax.experimental import pallas as pl


def kernel(timestamps, timestamp_weights, position_weights):
    raise NotImplementedError("write your pallas kernel here")


# speedup vs baseline: 1119.6887x; 1119.6887x over previous
import functools

import jax
import jax.numpy as jnp
from jax.experimental import pallas as pl
from jax.experimental.pallas import tpu as pltpu

_L = 200
_NB = 128
_G = 8


def _tc_body(ts_ref, tw_ref, pos_ref, out_ref):
    ts = ts_ref[...]
    nxt = jnp.concatenate([ts[:, 1:], ts[:, _L - 1 : _L]], axis=1)
    diff = nxt[:, :, None] - ts[:, None, :]
    m = jnp.maximum(jnp.abs(diff).astype(jnp.float32), 1.0)
    b = jnp.clip((jnp.log(m) / 0.301).astype(jnp.int32), 0, _NB - 1)
    table = jnp.broadcast_to(tw_ref[0][None, None, :], (_G, _L, _NB))
    w = jnp.take_along_axis(table, b, axis=-1, mode="promise_in_bounds")
    out_ref[...] = w + pos_ref[...]


def kernel(timestamps, timestamp_weights, position_weights):
    Bsz = timestamps.shape[0]
    t = jnp.concatenate(
        [position_weights[: 2 * _L - 1], jnp.zeros((_L,), dtype=position_weights.dtype)]
    )
    t = jnp.tile(t, _L)[: -_L].reshape(_L, 3 * _L - 2)
    r = (2 * _L - 1) // 2
    pos = t[:, r : 3 * _L - 2 - r]

    tw = timestamp_weights[: _NB].reshape(1, _NB)

    grid = (Bsz // _G,)
    out = pl.pallas_call(
        _tc_body,
        grid=grid,
        in_specs=[
            pl.BlockSpec((_G, _L), lambda i: (i, 0)),
            pl.BlockSpec((1, _NB), lambda i: (0, 0)),
            pl.BlockSpec((_L, _L), lambda i: (0, 0)),
        ],
        out_specs=pl.BlockSpec((_G, _L, _L), lambda i: (i, 0, 0)),
        out_shape=jax.ShapeDtypeStruct((Bsz, _L, _L), jnp.float32),
    )(timestamps, tw, pos)
    return out


# speedup vs baseline: 1157.8346x; 1.0341x over previous
import functools

import jax
import jax.numpy as jnp
from jax.experimental import pallas as pl
from jax.experimental.pallas import tpu as pltpu

_L = 200
_NB = 128
_G = 16


def _tc_body(ts_ref, tw_ref, pos_ref, out_ref):
    ts = ts_ref[...].astype(jnp.float32)
    nxt = jnp.concatenate([ts[:, 1:], ts[:, _L - 1 : _L]], axis=1)
    diff = nxt[:, :, None] - ts[:, None, :]
    m = jnp.maximum(jnp.abs(diff), 1.0)
    b = (jnp.log(m) * jnp.float32(1.0 / 0.301)).astype(jnp.int32)
    table = jnp.broadcast_to(tw_ref[0][None, None, :], (_G, _L, _NB))
    w = jnp.take_along_axis(table, b, axis=-1, mode="promise_in_bounds")
    out_ref[...] = w + pos_ref[...]


def kernel(timestamps, timestamp_weights, position_weights):
    Bsz = timestamps.shape[0]
    t = jnp.concatenate(
        [position_weights[: 2 * _L - 1], jnp.zeros((_L,), dtype=position_weights.dtype)]
    )
    t = jnp.tile(t, _L)[: -_L].reshape(_L, 3 * _L - 2)
    r = (2 * _L - 1) // 2
    pos = t[:, r : 3 * _L - 2 - r]

    tw = timestamp_weights[: _NB].reshape(1, _NB)

    grid = (Bsz // _G,)
    out = pl.pallas_call(
        _tc_body,
        grid=grid,
        in_specs=[
            pl.BlockSpec((_G, _L), lambda i: (i, 0)),
            pl.BlockSpec((1, _NB), lambda i: (0, 0)),
            pl.BlockSpec((_L, _L), lambda i: (0, 0)),
        ],
        out_specs=pl.BlockSpec((_G, _L, _L), lambda i: (i, 0, 0)),
        out_shape=jax.ShapeDtypeStruct((Bsz, _L, _L), jnp.float32),
    )(timestamps, tw, pos)
    return out


# speedup vs baseline: 1174.6539x; 1.0145x over previous
import functools

import jax
import jax.numpy as jnp
from jax.experimental import pallas as pl
from jax.experimental.pallas import tpu as pltpu

_L = 200
_NB = 128
_G = 32


def _tc_body(ts_ref, tw_ref, pos_ref, out_ref):
    ts = ts_ref[...].astype(jnp.float32)
    nxt = jnp.concatenate([ts[:, 1:], ts[:, _L - 1 : _L]], axis=1)
    diff = nxt[:, :, None] - ts[:, None, :]
    m = jnp.maximum(jnp.abs(diff), 1.0)
    b = (jnp.log(m) * jnp.float32(1.0 / 0.301)).astype(jnp.int32)
    table = jnp.broadcast_to(tw_ref[0][None, None, :], (_G, _L, _NB))
    w = jnp.take_along_axis(table, b, axis=-1, mode="promise_in_bounds")
    out_ref[...] = w + pos_ref[...]


def kernel(timestamps, timestamp_weights, position_weights):
    Bsz = timestamps.shape[0]
    t = jnp.concatenate(
        [position_weights[: 2 * _L - 1], jnp.zeros((_L,), dtype=position_weights.dtype)]
    )
    t = jnp.tile(t, _L)[: -_L].reshape(_L, 3 * _L - 2)
    r = (2 * _L - 1) // 2
    pos = t[:, r : 3 * _L - 2 - r]

    tw = timestamp_weights[: _NB].reshape(1, _NB)

    grid = (Bsz // _G,)
    out = pl.pallas_call(
        _tc_body,
        grid=grid,
        in_specs=[
            pl.BlockSpec((_G, _L), lambda i: (i, 0)),
            pl.BlockSpec((1, _NB), lambda i: (0, 0)),
            pl.BlockSpec((_L, _L), lambda i: (0, 0)),
        ],
        out_specs=pl.BlockSpec((_G, _L, _L), lambda i: (i, 0, 0)),
        out_shape=jax.ShapeDtypeStruct((Bsz, _L, _L), jnp.float32),
    )(timestamps, tw, pos)
    return out


# speedup vs baseline: 1177.2762x; 1.0022x over previous
import functools

import jax
import jax.numpy as jnp
from jax.experimental import pallas as pl
from jax.experimental.pallas import tpu as pltpu

_L = 200
_NB = 128
_G = 64


def _tc_body(ts_ref, tw_ref, pos_ref, out_ref):
    ts = ts_ref[...].astype(jnp.float32)
    nxt = jnp.concatenate([ts[:, 1:], ts[:, _L - 1 : _L]], axis=1)
    diff = nxt[:, :, None] - ts[:, None, :]
    m = jnp.maximum(jnp.abs(diff), 1.0)
    b = (jnp.log(m) * jnp.float32(1.0 / 0.301)).astype(jnp.int32)
    table = jnp.broadcast_to(tw_ref[0][None, None, :], (_G, _L, _NB))
    w = jnp.take_along_axis(table, b, axis=-1, mode="promise_in_bounds")
    out_ref[...] = w + pos_ref[...]


def kernel(timestamps, timestamp_weights, position_weights):
    Bsz = timestamps.shape[0]
    t = jnp.concatenate(
        [position_weights[: 2 * _L - 1], jnp.zeros((_L,), dtype=position_weights.dtype)]
    )
    t = jnp.tile(t, _L)[: -_L].reshape(_L, 3 * _L - 2)
    r = (2 * _L - 1) // 2
    pos = t[:, r : 3 * _L - 2 - r]

    tw = timestamp_weights[: _NB].reshape(1, _NB)

    grid = (Bsz // _G,)
    out = pl.pallas_call(
        _tc_body,
        grid=grid,
        in_specs=[
            pl.BlockSpec((_G, _L), lambda i: (i, 0)),
            pl.BlockSpec((1, _NB), lambda i: (0, 0)),
            pl.BlockSpec((_L, _L), lambda i: (0, 0)),
        ],
        out_specs=pl.BlockSpec((_G, _L, _L), lambda i: (i, 0, 0)),
        out_shape=jax.ShapeDtypeStruct((Bsz, _L, _L), jnp.float32),
    )(timestamps, tw, pos)
    return out
